# baseline (speedup 1.0000x reference)
import functools

import jax
import jax.numpy as jnp
import numpy as np
from jax import lax
from jax.experimental import pallas as pl
from jax.experimental.pallas import tpu as pltpu
from jax.experimental.pallas import tpu_sc as plsc

N = 10000
D = 128
E = 320000
NC = 2
NS = 16
LANES = 16
DH = D // NC

CH = 128
E2 = 327680
PAD = E2 - E
EPT = E2 // NS
NCHUNK = EPT // CH
CPS = 8
SCH = CH * CPS
NSUP = NCHUNK // CPS
NB = 4
STRIPE = 632
ZB = 16

_mesh = plsc.VectorSubcoreMesh(core_axis_name="c", subcore_axis_name="s")
_GDN = jax.lax.GatherDimensionNumbers(
    offset_dims=(), collapsed_slice_dims=(0,), start_index_map=(0,))


@functools.partial(
    pl.kernel,
    out_type=jax.ShapeDtypeStruct((NC * N, DH), jnp.float32),
    mesh=_mesh,
    scratch_types=[
        pltpu.VMEM((2, SCH), jnp.int32),
        pltpu.VMEM((2, CPS, CH), jnp.int32),
        pltpu.VMEM((2, SCH), jnp.float32),
        pltpu.VMEM((CH, DH // 2), jnp.int32),
        pltpu.VMEM((CH, DH // 2), jnp.int32),
        pltpu.VMEM((CH, DH), jnp.float32),
        pltpu.VMEM((CH, DH), jnp.float32),
        pltpu.VMEM((ZB, DH), jnp.float32),
        pltpu.VMEM_SHARED((N, DH), jnp.float32),
        pltpu.SemaphoreType.DMA,
        pltpu.SemaphoreType.DMA,
        pltpu.SemaphoreType.DMA,
        pltpu.SemaphoreType.DMA,
        pltpu.SemaphoreType.DMA,
    ],
    compiler_params=pltpu.CompilerParams(needs_layout_passes=False,
                                         use_tc_tiling_on_sc=False),
)
def _sc_edge_kernel(h_hbm, src_hbm, dst_hbm, w_hbm, out_hbm,
                    srcs, dsts, ws, gr0, gr1, mr0, mr1, zbuf, acc,
                    gsem0, gsem1, ssem0, ssem1, isem):
    c = lax.axis_index("c")
    s = lax.axis_index("s")
    grows = (gr0, gr1)
    mrows = (mr0, mr1)
    gsems = (gsem0, gsem1)
    ssems = (ssem0, ssem1)

    @pl.loop(0, ZB)
    def _zero(r):
        for g in range(DH // LANES):
            zbuf[r, pl.ds(g * LANES, LANES)] = jnp.zeros((LANES,), jnp.float32)

    row0 = jnp.minimum(s * STRIPE, N - STRIPE)

    @pl.loop(0, STRIPE // ZB)
    def _zfill(z):
        pltpu.sync_copy(zbuf, acc.at[pl.ds(row0 + z * ZB, ZB)])
    pltpu.sync_copy(zbuf.at[pl.ds(0, STRIPE % ZB)],
                    acc.at[pl.ds(row0 + (STRIPE // ZB) * ZB, STRIPE % ZB)])

    ebase = s * EPT

    def _idx_stage(S, sb):
        off = ebase + S * SCH
        row = (ebase + S * SCH) // CH
        return [
            pltpu.make_async_copy(src_hbm.at[pl.ds(off, SCH)], srcs.at[sb], isem),
            pltpu.make_async_copy(dst_hbm.at[pl.ds(row, CPS)], dsts.at[sb], isem),
            pltpu.make_async_copy(w_hbm.at[pl.ds(off, SCH)], ws.at[sb], isem),
        ]

    def _gather(kk, b, sb):
        return pltpu.make_async_copy(
            h_hbm.at[c].at[srcs.at[sb].at[pl.ds(kk * CH, CH)]],
            grows[b], gsems[b])

    def _scatter(kk, b, sb):
        return pltpu.make_async_copy(
            mrows[b], acc.at[dsts.at[sb].at[kk]], ssems[b])

    def _stage(kk, b, sb):
        off = kk * CH
        _gather(kk, b, sb).wait()

        @pl.when(kk >= 2)
        def _drain():
            _scatter(kk - 2, b, sb).wait()

        @pl.loop(0, CH, step=LANES)
        def _row16(r0):
            wchunk = ws[sb, pl.ds(off + r0, LANES)]
            for rr in range(LANES):
                wv = lax.gather(wchunk,
                                jnp.broadcast_to(rr, (LANES, 1)).astype(jnp.int32),
                                _GDN, (1,),
                                mode=lax.GatherScatterMode.PROMISE_IN_BOUNDS)
                for g in range(DH // 32):
                    w32 = grows[b][r0 + rr, pl.ds(g * LANES, LANES)]
                    lo = plsc.bitcast(lax.shift_left(w32, 16), jnp.float32)
                    hi = plsc.bitcast(
                        jnp.bitwise_and(w32, jnp.int32(-65536)), jnp.float32)
                    mrows[b][r0 + rr, pl.ds(32 * g, LANES)] = lo * wv
                    mrows[b][r0 + rr, pl.ds(32 * g + LANES, LANES)] = hi * wv

        pltpu.async_copy(mrows[b], acc.at[dsts.at[sb].at[kk]], ssems[b],
                         add=True)

        @pl.when(kk + 2 < CPS)
        def _refill():
            _gather(kk + 2, b, sb).start()

    for d in _idx_stage(0, 0):
        d.start()
    for d in _idx_stage(0, 0):
        d.wait()
    _gather(jnp.int32(0), 0, 0).start()
    _gather(jnp.int32(1), 1, 0).start()
    for d in _idx_stage(1, 1):
        d.start()

    @pl.loop(0, NSUP)
    def _sup(S):
        sb = S % 2

        @pl.loop(0, CPS // 2)
        def _pair(p):
            for b in range(2):
                _stage(2 * p + b, b, sb)

        @pl.when(S + 1 < NSUP)
        def _next():
            for d in _idx_stage(S + 1, 1 - sb):
                d.wait()
            _gather(jnp.int32(0), 0, 1 - sb).start()
            _gather(jnp.int32(1), 1, 1 - sb).start()

        _scatter(jnp.int32(CPS - 2), 0, sb).wait()
        _scatter(jnp.int32(CPS - 1), 1, sb).wait()

        @pl.when(S + 2 < NSUP)
        def _pf():
            for d in _idx_stage(S + 2, sb):
                d.start()

    plsc.subcore_barrier()

    pltpu.sync_copy(acc.at[pl.ds(row0, STRIPE)],
                    out_hbm.at[pl.ds(c * N + row0, STRIPE)])


BR = 2000


def _pack_half(h64):
    words = []
    for g in range(DH // 32):
        a = h64[:, 32 * g:32 * g + 16]
        b = h64[:, 32 * g + 16:32 * g + 32]
        au = lax.convert_element_type(
            lax.bitcast_convert_type(a.astype(jnp.bfloat16), jnp.uint16),
            jnp.uint32)
        bu = lax.convert_element_type(
            lax.bitcast_convert_type(b.astype(jnp.bfloat16), jnp.uint16),
            jnp.uint32)
        words.append(lax.bitcast_convert_type(
            au | (bu << jnp.uint32(16)), jnp.int32))
    return jnp.concatenate(words, axis=1)


def _split(h, h_ref):
    h_ref[0] = _pack_half(h[:, :DH])
    h_ref[1] = _pack_half(h[:, DH:])


def _tc_first_body(x_ref, w_ref, b_ref, t_ref, h_ref, hid_ref):
    x = x_ref[...]
    hid_ref[...] = t_ref[0, 0] * x
    _split(lax.dot_general(x, w_ref[...], (((1,), (1,)), ((), ())),
                           preferred_element_type=jnp.float32) + b_ref[...],
           h_ref)


def _tc_mid_body(p_ref, hid_ref, w_ref, b_ref, t_ref, h_ref, hidout_ref):
    xi = jnp.maximum(jnp.concatenate([p_ref[0], p_ref[1]], axis=1), 0.0)
    hidout_ref[...] = hid_ref[...] + t_ref[0, 0] * xi
    _split(lax.dot_general(xi, w_ref[...], (((1,), (1,)), ((), ())),
                           preferred_element_type=jnp.float32) + b_ref[...],
           h_ref)


def _tc_last_body(p_ref, hid_ref, t_ref, hidout_ref):
    xi = jnp.maximum(jnp.concatenate([p_ref[0], p_ref[1]], axis=1), 0.0)
    hidout_ref[...] = hid_ref[...] + t_ref[0, 0] * xi


_xD = pl.BlockSpec((BR, D), lambda i: (i, 0))
_pD = pl.BlockSpec((2, BR, DH), lambda i: (0, i, 0))
_hD = pl.BlockSpec((2, BR, DH // 2), lambda i: (0, i, 0))
_wD = pl.BlockSpec((D, D), lambda i: (0, 0))
_bD = pl.BlockSpec((1, D), lambda i: (0, 0))
_tD = pl.BlockSpec((1, 1), lambda i: (0, 0))
_GRID = (N // BR,)
_fND = jax.ShapeDtypeStruct((N, D), jnp.float32)
_fSplit = jax.ShapeDtypeStruct((NC, N, DH // 2), jnp.int32)

_tc_first = pl.pallas_call(
    _tc_first_body,
    grid=_GRID,
    in_specs=[_xD, _wD, _bD, _tD],
    out_specs=[_hD, _xD],
    out_shape=[_fSplit, _fND],
)

_tc_mid = pl.pallas_call(
    _tc_mid_body,
    grid=_GRID,
    in_specs=[_pD, _xD, _wD, _bD, _tD],
    out_specs=[_hD, _xD],
    out_shape=[_fSplit, _fND],
)

_tc_last = pl.pallas_call(
    _tc_last_body,
    grid=_GRID,
    in_specs=[_pD, _xD, _tD],
    out_specs=_xD,
    out_shape=_fND,
)


def kernel(x, edge_index, edge_weight, W0, b0, W1, b1, W2, b2, W3, b3, temp):
    ipad = jnp.arange(PAD, dtype=jnp.int32) % N
    src = jnp.concatenate([edge_index[0], ipad])
    dst = jnp.concatenate([edge_index[1], ipad]).reshape(E2 // CH, CH)
    w = jnp.concatenate([edge_weight, jnp.zeros((PAD,), jnp.float32)])
    params = [(W0, b0), (W1, b1), (W2, b2), (W3, b3)]

    h, hidden = _tc_first(x, W0, b0.reshape(1, D), temp[0].reshape(1, 1))
    for i in range(4):
        parts = _sc_edge_kernel(h, src, dst, w).reshape(NC, N, DH)
        if i < 3:
            W, b = params[i + 1]
            h, hidden = _tc_mid(parts, hidden, W, b.reshape(1, D),
                                temp[i + 1].reshape(1, 1))
        else:
            hidden = _tc_last(parts, hidden, temp[i + 1].reshape(1, 1))
    return hidden


# speedup vs baseline: 1.8810x; 1.8810x over previous
import functools

import jax
import jax.numpy as jnp
import numpy as np
from jax import lax
from jax.experimental import pallas as pl
from jax.experimental.pallas import tpu as pltpu
from jax.experimental.pallas import tpu_sc as plsc

N = 10000
D = 128
E = 320000
NC = 2
NS = 16
LANES = 16
DH = D // NC

CH = 128
E2 = 327680
PAD = E2 - E
EPT = E2 // NS
NCHUNK = EPT // CH
CPS = 8
SCH = CH * CPS
NSUP = NCHUNK // CPS
NB = 4
STRIPE = 632
ZB = 16

_mesh = plsc.VectorSubcoreMesh(core_axis_name="c", subcore_axis_name="s")
_GDN = jax.lax.GatherDimensionNumbers(
    offset_dims=(), collapsed_slice_dims=(0,), start_index_map=(0,))


@functools.partial(
    pl.kernel,
    out_type=jax.ShapeDtypeStruct((NC * N, DH), jnp.float32),
    mesh=_mesh,
    scratch_types=[
        pltpu.VMEM((2, SCH), jnp.int32),
        pltpu.VMEM((2, CPS, CH), jnp.int32),
        pltpu.VMEM((2, SCH), jnp.float32),
        pltpu.VMEM((CH, DH // 2), jnp.int32),
        pltpu.VMEM((CH, DH // 2), jnp.int32),
        pltpu.VMEM((CH, DH), jnp.float32),
        pltpu.VMEM((CH, DH), jnp.float32),
        pltpu.VMEM((ZB, DH), jnp.float32),
        pltpu.VMEM_SHARED((N, DH), jnp.float32),
        pltpu.SemaphoreType.DMA,
        pltpu.SemaphoreType.DMA,
        pltpu.SemaphoreType.DMA,
        pltpu.SemaphoreType.DMA,
        pltpu.SemaphoreType.DMA,
    ],
    compiler_params=pltpu.CompilerParams(needs_layout_passes=False,
                                         use_tc_tiling_on_sc=False),
)
def _sc_edge_kernel(h_hbm, src_hbm, dst_hbm, w_hbm, out_hbm,
                    srcs, dsts, ws, gr0, gr1, mr0, mr1, zbuf, acc,
                    gsem0, gsem1, ssem0, ssem1, isem):
    c = lax.axis_index("c")
    s = lax.axis_index("s")
    grows = (gr0, gr1)
    mrows = (mr0, mr1)
    gsems = (gsem0, gsem1)
    ssems = (ssem0, ssem1)

    @pl.loop(0, ZB)
    def _zero(r):
        for g in range(DH // LANES):
            zbuf[r, pl.ds(g * LANES, LANES)] = jnp.zeros((LANES,), jnp.float32)

    row0 = jnp.minimum(s * STRIPE, N - STRIPE)

    @pl.loop(0, STRIPE // ZB)
    def _zfill(z):
        pltpu.sync_copy(zbuf, acc.at[pl.ds(row0 + z * ZB, ZB)])
    pltpu.sync_copy(zbuf.at[pl.ds(0, STRIPE % ZB)],
                    acc.at[pl.ds(row0 + (STRIPE // ZB) * ZB, STRIPE % ZB)])

    ebase = s * EPT

    def _idx_stage(S, sb):
        off = ebase + S * SCH
        row = (ebase + S * SCH) // CH
        return [
            pltpu.make_async_copy(src_hbm.at[pl.ds(off, SCH)], srcs.at[sb], isem),
            pltpu.make_async_copy(dst_hbm.at[pl.ds(row, CPS)], dsts.at[sb], isem),
            pltpu.make_async_copy(w_hbm.at[pl.ds(off, SCH)], ws.at[sb], isem),
        ]

    def _gather(kk, b, sb):
        return pltpu.make_async_copy(
            h_hbm.at[c].at[srcs.at[sb].at[pl.ds(kk * CH, CH)]],
            grows[b], gsems[b])

    def _scatter(kk, b, sb):
        return pltpu.make_async_copy(
            mrows[b], acc.at[dsts.at[sb].at[kk]], ssems[b])

    def _stage(kk, b, sb):
        off = kk * CH
        _gather(kk, b, sb).wait()

        @pl.when(kk >= 2)
        def _drain():
            _scatter(kk - 2, b, sb).wait()

        @plsc.parallel_loop(0, CH, step=LANES, unroll=2)
        def _row16(r0):
            wchunk = ws[sb, pl.ds(off + r0, LANES)]
            for rr in range(LANES):
                wv = lax.gather(wchunk,
                                jnp.broadcast_to(rr, (LANES, 1)).astype(jnp.int32),
                                _GDN, (1,),
                                mode=lax.GatherScatterMode.PROMISE_IN_BOUNDS)
                for g in range(DH // 32):
                    w32 = grows[b][r0 + rr, pl.ds(g * LANES, LANES)]
                    lo = plsc.bitcast(lax.shift_left(w32, 16), jnp.float32)
                    hi = plsc.bitcast(
                        jnp.bitwise_and(w32, jnp.int32(-65536)), jnp.float32)
                    mrows[b][r0 + rr, pl.ds(32 * g, LANES)] = lo * wv
                    mrows[b][r0 + rr, pl.ds(32 * g + LANES, LANES)] = hi * wv

        pltpu.async_copy(mrows[b], acc.at[dsts.at[sb].at[kk]], ssems[b],
                         add=True)

        @pl.when(kk + 2 < CPS)
        def _refill():
            _gather(kk + 2, b, sb).start()

    for d in _idx_stage(0, 0):
        d.start()
    for d in _idx_stage(0, 0):
        d.wait()
    _gather(jnp.int32(0), 0, 0).start()
    _gather(jnp.int32(1), 1, 0).start()
    for d in _idx_stage(1, 1):
        d.start()

    @pl.loop(0, NSUP)
    def _sup(S):
        sb = S % 2

        @pl.loop(0, CPS // 2)
        def _pair(p):
            for b in range(2):
                _stage(2 * p + b, b, sb)

        @pl.when(S + 1 < NSUP)
        def _next():
            for d in _idx_stage(S + 1, 1 - sb):
                d.wait()
            _gather(jnp.int32(0), 0, 1 - sb).start()
            _gather(jnp.int32(1), 1, 1 - sb).start()

        _scatter(jnp.int32(CPS - 2), 0, sb).wait()
        _scatter(jnp.int32(CPS - 1), 1, sb).wait()

        @pl.when(S + 2 < NSUP)
        def _pf():
            for d in _idx_stage(S + 2, sb):
                d.start()

    plsc.subcore_barrier()

    pltpu.sync_copy(acc.at[pl.ds(row0, STRIPE)],
                    out_hbm.at[pl.ds(c * N + row0, STRIPE)])


BR = 2000


def _pack_half(h64):
    words = []
    for g in range(DH // 32):
        a = h64[:, 32 * g:32 * g + 16]
        b = h64[:, 32 * g + 16:32 * g + 32]
        au = lax.convert_element_type(
            lax.bitcast_convert_type(a.astype(jnp.bfloat16), jnp.uint16),
            jnp.uint32)
        bu = lax.convert_element_type(
            lax.bitcast_convert_type(b.astype(jnp.bfloat16), jnp.uint16),
            jnp.uint32)
        words.append(lax.bitcast_convert_type(
            au | (bu << jnp.uint32(16)), jnp.int32))
    return jnp.concatenate(words, axis=1)


def _split(h, h_ref):
    h_ref[0] = _pack_half(h[:, :DH])
    h_ref[1] = _pack_half(h[:, DH:])


def _tc_first_body(x_ref, w_ref, b_ref, t_ref, h_ref, hid_ref):
    x = x_ref[...]
    hid_ref[...] = t_ref[0, 0] * x
    _split(lax.dot_general(x, w_ref[...], (((1,), (1,)), ((), ())),
                           preferred_element_type=jnp.float32) + b_ref[...],
           h_ref)


def _tc_mid_body(p_ref, hid_ref, w_ref, b_ref, t_ref, h_ref, hidout_ref):
    xi = jnp.maximum(jnp.concatenate([p_ref[0], p_ref[1]], axis=1), 0.0)
    hidout_ref[...] = hid_ref[...] + t_ref[0, 0] * xi
    _split(lax.dot_general(xi, w_ref[...], (((1,), (1,)), ((), ())),
                           preferred_element_type=jnp.float32) + b_ref[...],
           h_ref)


def _tc_last_body(p_ref, hid_ref, t_ref, hidout_ref):
    xi = jnp.maximum(jnp.concatenate([p_ref[0], p_ref[1]], axis=1), 0.0)
    hidout_ref[...] = hid_ref[...] + t_ref[0, 0] * xi


_xD = pl.BlockSpec((BR, D), lambda i: (i, 0))
_pD = pl.BlockSpec((2, BR, DH), lambda i: (0, i, 0))
_hD = pl.BlockSpec((2, BR, DH // 2), lambda i: (0, i, 0))
_wD = pl.BlockSpec((D, D), lambda i: (0, 0))
_bD = pl.BlockSpec((1, D), lambda i: (0, 0))
_tD = pl.BlockSpec((1, 1), lambda i: (0, 0))
_GRID = (N // BR,)
_fND = jax.ShapeDtypeStruct((N, D), jnp.float32)
_fSplit = jax.ShapeDtypeStruct((NC, N, DH // 2), jnp.int32)

_tc_first = pl.pallas_call(
    _tc_first_body,
    grid=_GRID,
    in_specs=[_xD, _wD, _bD, _tD],
    out_specs=[_hD, _xD],
    out_shape=[_fSplit, _fND],
)

_tc_mid = pl.pallas_call(
    _tc_mid_body,
    grid=_GRID,
    in_specs=[_pD, _xD, _wD, _bD, _tD],
    out_specs=[_hD, _xD],
    out_shape=[_fSplit, _fND],
)

_tc_last = pl.pallas_call(
    _tc_last_body,
    grid=_GRID,
    in_specs=[_pD, _xD, _tD],
    out_specs=_xD,
    out_shape=_fND,
)


def kernel(x, edge_index, edge_weight, W0, b0, W1, b1, W2, b2, W3, b3, temp):
    ipad = jnp.arange(PAD, dtype=jnp.int32) % N
    src = jnp.concatenate([edge_index[0], ipad])
    dst = jnp.concatenate([edge_index[1], ipad]).reshape(E2 // CH, CH)
    w = jnp.concatenate([edge_weight, jnp.zeros((PAD,), jnp.float32)])
    params = [(W0, b0), (W1, b1), (W2, b2), (W3, b3)]

    h, hidden = _tc_first(x, W0, b0.reshape(1, D), temp[0].reshape(1, 1))
    for i in range(4):
        parts = _sc_edge_kernel(h, src, dst, w).reshape(NC, N, DH)
        if i < 3:
            W, b = params[i + 1]
            h, hidden = _tc_mid(parts, hidden, W, b.reshape(1, D),
                                temp[i + 1].reshape(1, 1))
        else:
            hidden = _tc_last(parts, hidden, temp[i + 1].reshape(1, 1))
    return hidden
